# baseline (speedup 1.0000x reference)
import jax
import jax.numpy as jnp
from jax import lax
from jax.experimental import pallas as pl
from jax.experimental.pallas import tpu as pltpu

_B = 8192
_D = 768
_H = 256
_C = 2
_E = 8
_GH = 128
_BT = 512


def _moe_body(x_ref, gw1_ref, gb1_ref, gw2_ref, gb2_ref,
              We1_ref, be1_ref, ln_g_ref, ln_b_ref, We2_ref, be2_ref,
              scores_ref, idx_ref, out8_ref,
              p_acc, sc_acc, w1b_ref):
    step = pl.program_id(0)
    nsteps = pl.num_programs(0)
    xb = x_ref[...]

    g1 = jnp.dot(xb, gw1_ref[...], preferred_element_type=jnp.float32)
    g1 = jnp.maximum(g1 + gb1_ref[...], 0.0)
    logits = jnp.dot(g1, gw2_ref[...], preferred_element_type=jnp.float32)
    logits = logits + gb2_ref[...]
    mx = jnp.max(logits, axis=-1, keepdims=True)
    ex = jnp.exp(logits - mx)
    scores = ex / jnp.sum(ex, axis=-1, keepdims=True)
    scores_ref[...] = scores

    eiota = lax.broadcasted_iota(jnp.int32, (_BT, _E), 1)
    m1 = jnp.max(scores, axis=-1, keepdims=True)
    i1 = jnp.min(jnp.where(scores == m1, eiota, _E), axis=-1, keepdims=True)
    masked = jnp.where(eiota == i1, -1.0, scores)
    m2 = jnp.max(masked, axis=-1, keepdims=True)
    i2 = jnp.min(jnp.where(masked == m2, eiota, _E), axis=-1, keepdims=True)
    idx_ref[...] = jnp.concatenate([i1, i2], axis=1)
    denom = m1 + m2
    w1 = m1 / denom
    w2 = m2 / denom
    gates = jnp.where(eiota == i1, w1, 0.0) + jnp.where(eiota == i2, w2, 0.0)

    @pl.when(step == 0)
    def _init():
        p_acc[...] = jnp.zeros_like(p_acc)
        for e in range(_E):
            sc_acc[0, e] = 0.0
            sc_acc[1, e] = 0.0
        w1b_ref[...] = We1_ref[...].astype(jnp.bfloat16)

    xb16 = xb.astype(jnp.bfloat16)

    for e in range(_E):
        h = jnp.dot(xb16, w1b_ref[e], preferred_element_type=jnp.float32)
        h = h + be1_ref[e:e + 1, :]
        h = 0.5 * h * (1.0 + lax.erf(h * 0.70710678118654752))
        sh = jnp.sum(h, axis=-1, keepdims=True)
        shh = jnp.sum(h * h, axis=-1, keepdims=True)
        mu = sh * (1.0 / _H)
        var = shh * (1.0 / _H) - mu * mu
        rstd = lax.rsqrt(var + 1e-5)
        ge = gates[:, e:e + 1]
        a = ge * rstd
        p_acc[e:e + 1, :] += jnp.sum(a * h, axis=0, keepdims=True)
        sc_acc[0, e] += jnp.sum(a * mu)
        sc_acc[1, e] += jnp.sum(ge)

    @pl.when(step == nsteps - 1)
    def _finish():
        for e in range(_E):
            q = sc_acc[0, e]
            r = sc_acc[1, e]
            s = (ln_g_ref[e:e + 1, :] * (p_acc[e:e + 1, :] - q)
                 + ln_b_ref[e:e + 1, :] * r)
            o = jnp.dot(s, We2_ref[e], preferred_element_type=jnp.float32)
            out8_ref[e:e + 1, :] = o + be2_ref[e:e + 1, :] * r


def kernel(x, gw1, gb1, gw2, gb2, We1, be1, ln_g, ln_b, We2, be2):
    nsteps = _B // _BT
    full = lambda i: (0, 0)
    full3 = lambda i: (0, 0, 0)
    scores, idx, out8 = pl.pallas_call(
        _moe_body,
        grid=(nsteps,),
        in_specs=[
            pl.BlockSpec((_BT, _D), lambda i: (i, 0)),
            pl.BlockSpec((_D, _GH), full),
            pl.BlockSpec((1, _GH), full),
            pl.BlockSpec((_GH, _E), full),
            pl.BlockSpec((1, _E), full),
            pl.BlockSpec((_E, _D, _H), full3),
            pl.BlockSpec((_E, _H), full),
            pl.BlockSpec((_E, _H), full),
            pl.BlockSpec((_E, _H), full),
            pl.BlockSpec((_E, _H, _C), full3),
            pl.BlockSpec((_E, _C), full),
        ],
        out_specs=[
            pl.BlockSpec((_BT, _E), lambda i: (i, 0)),
            pl.BlockSpec((_BT, 2), lambda i: (i, 0)),
            pl.BlockSpec((_E, _C), full),
        ],
        out_shape=[
            jax.ShapeDtypeStruct((_B, _E), jnp.float32),
            jax.ShapeDtypeStruct((_B, 2), jnp.int32),
            jax.ShapeDtypeStruct((_E, _C), jnp.float32),
        ],
        scratch_shapes=[
            pltpu.VMEM((_E, _H), jnp.float32),
            pltpu.SMEM((2, _E), jnp.float32),
            pltpu.VMEM((_E, _D, _H), jnp.bfloat16),
        ],
    )(x, gw1, gb1.reshape(1, _GH), gw2, gb2.reshape(1, _E),
      We1, be1, ln_g, ln_b, We2, be2)
    output = jnp.zeros((_B, _C), jnp.float32).at[:_E, :].set(out8)
    return output, scores, idx


# speedup vs baseline: 1.5031x; 1.5031x over previous
import jax
import jax.numpy as jnp
from jax import lax
from jax.experimental import pallas as pl
from jax.experimental.pallas import tpu as pltpu

_B = 8192
_D = 768
_H = 256
_C = 2
_E = 8
_GH = 128
_BT = 512
_W = _GH + _E * _H


def _moe_body(x_ref, gw1_ref, gb1_ref, gw2_ref, gb2_ref,
              We1_ref, be1_ref, ln_g_ref, ln_b_ref, We2_ref, be2_ref,
              scores_ref, idx_ref, out8_ref,
              p_acc, r_acc, wall_ref, ball_ref):
    step = pl.program_id(0)
    nsteps = pl.num_programs(0)

    @pl.when(step == 0)
    def _init():
        p_acc[...] = jnp.zeros_like(p_acc)
        r_acc[...] = jnp.zeros_like(r_acc)
        wall_ref[:, :_GH] = gw1_ref[...]
        ball_ref[:, :_GH] = gb1_ref[...]
        for e in range(_E):
            lo = _GH + e * _H
            wall_ref[:, lo:lo + _H] = We1_ref[e]
            ball_ref[:, lo:lo + _H] = be1_ref[e:e + 1, :]

    xb = x_ref[...]
    hall = jnp.dot(xb, wall_ref[...], preferred_element_type=jnp.float32)
    hall = hall + ball_ref[...]

    g1 = jnp.maximum(hall[:, :_GH], 0.0)
    logits = jnp.dot(g1, gw2_ref[...], preferred_element_type=jnp.float32)
    logits = logits + gb2_ref[...]
    mx = jnp.max(logits, axis=-1, keepdims=True)
    ex = jnp.exp(logits - mx)
    scores = ex / jnp.sum(ex, axis=-1, keepdims=True)
    scores_ref[...] = scores

    eiota = lax.broadcasted_iota(jnp.int32, (_BT, _E), 1)
    m1 = jnp.max(scores, axis=-1, keepdims=True)
    i1 = jnp.min(jnp.where(scores == m1, eiota, _E), axis=-1, keepdims=True)
    masked = jnp.where(eiota == i1, -1.0, scores)
    m2 = jnp.max(masked, axis=-1, keepdims=True)
    i2 = jnp.min(jnp.where(masked == m2, eiota, _E), axis=-1, keepdims=True)
    idx_ref[...] = jnp.concatenate([i1, i2], axis=1)
    rd = 1.0 / (m1 + m2)
    w1 = m1 * rd
    w2 = m2 * rd
    gates = jnp.where(eiota == i1, w1, 0.0) + jnp.where(eiota == i2, w2, 0.0)
    r_acc[...] += jnp.sum(gates, axis=0, keepdims=True)

    for e in range(_E):
        lo = _GH + e * _H
        h = hall[:, lo:lo + _H]
        h = 0.5 * h * (1.0 + lax.erf(h * 0.70710678118654752))
        mu = jnp.mean(h, axis=-1, keepdims=True)
        cen = h - mu
        var = jnp.mean(cen * cen, axis=-1, keepdims=True)
        ge = gates[:, e:e + 1]
        a = ge * lax.rsqrt(var + 1e-5)
        p_acc[e:e + 1, :] += jnp.sum(a * cen, axis=0, keepdims=True)

    @pl.when(step == nsteps - 1)
    def _finish():
        for e in range(_E):
            r = jnp.broadcast_to(r_acc[0:1, e:e + 1], (1, _H))
            s = ln_g_ref[e:e + 1, :] * p_acc[e:e + 1, :] + ln_b_ref[e:e + 1, :] * r
            o = jnp.dot(s, We2_ref[e], preferred_element_type=jnp.float32)
            out8_ref[e:e + 1, :] = o + be2_ref[e:e + 1, :] * r[:, :_C]


def kernel(x, gw1, gb1, gw2, gb2, We1, be1, ln_g, ln_b, We2, be2):
    nsteps = _B // _BT
    full = lambda i: (0, 0)
    full3 = lambda i: (0, 0, 0)
    scores, idx, out8 = pl.pallas_call(
        _moe_body,
        grid=(nsteps,),
        in_specs=[
            pl.BlockSpec((_BT, _D), lambda i: (i, 0)),
            pl.BlockSpec((_D, _GH), full),
            pl.BlockSpec((1, _GH), full),
            pl.BlockSpec((_GH, _E), full),
            pl.BlockSpec((1, _E), full),
            pl.BlockSpec((_E, _D, _H), full3),
            pl.BlockSpec((_E, _H), full),
            pl.BlockSpec((_E, _H), full),
            pl.BlockSpec((_E, _H), full),
            pl.BlockSpec((_E, _H, _C), full3),
            pl.BlockSpec((_E, _C), full),
        ],
        out_specs=[
            pl.BlockSpec((_BT, _E), lambda i: (i, 0)),
            pl.BlockSpec((_BT, 2), lambda i: (i, 0)),
            pl.BlockSpec((_E, _C), full),
        ],
        out_shape=[
            jax.ShapeDtypeStruct((_B, _E), jnp.float32),
            jax.ShapeDtypeStruct((_B, 2), jnp.int32),
            jax.ShapeDtypeStruct((_E, _C), jnp.float32),
        ],
        scratch_shapes=[
            pltpu.VMEM((_E, _H), jnp.float32),
            pltpu.VMEM((1, _E), jnp.float32),
            pltpu.VMEM((_D, _W), jnp.float32),
            pltpu.VMEM((1, _W), jnp.float32),
        ],
    )(x, gw1, gb1.reshape(1, _GH), gw2, gb2.reshape(1, _E),
      We1, be1, ln_g, ln_b, We2, be2)
    output = jnp.zeros((_B, _C), jnp.float32).at[:_E, :].set(out8)
    return output, scores, idx
